# Initial kernel scaffold; baseline (speedup 1.0000x reference)
import jax
import jax.numpy as jnp
from jax.experimental import pallas as pl


def kernel(z, triples, rel_weight):
    raise NotImplementedError("write your pallas kernel here")


# speedup vs baseline: 1.5645x; 1.5645x over previous
import functools

import jax
import jax.numpy as jnp
from jax import lax
from jax.experimental import pallas as pl
from jax.experimental.pallas import tpu as pltpu
from jax.experimental.pallas import tpu_sc as plsc

DIM = 128
BATCH = 16384

_INFO = plsc.get_sparse_core_info()
_NC, _NS, _L = _INFO.num_cores, _INFO.num_subcores, _INFO.num_lanes
_NW = _NC * _NS
_BPW = BATCH // _NW
_C = 128
_NCHUNK = _BPW // _C
_GROUPS = _C // _L
_NBUF = 2


def _sc_body(z_hbm, hidx_hbm, ridx_hbm, tidx_hbm, rel_hbm, out_hbm,
             idx_bufs, row_bufs, sems, stage_v, out_v):
    wid = lax.axis_index("s") * _NC + lax.axis_index("c")
    base = wid * _BPW

    lane = lax.broadcasted_iota(jnp.int32, (_L,), 0)
    lane16 = lane * _L

    descs = {}

    def issue(chunk, p):
        hidx_v, ridx_v, tidx_v = idx_bufs[p]
        hrows_v, wrows_v, trows_v = row_bufs[p]
        cbase = base + chunk * _C
        pltpu.sync_copy(hidx_hbm.at[pl.ds(cbase, _C)], hidx_v)
        pltpu.sync_copy(ridx_hbm.at[pl.ds(cbase, _C)], ridx_v)
        pltpu.sync_copy(tidx_hbm.at[pl.ds(cbase, _C)], tidx_v)
        descs[p] = (
            pltpu.async_copy(z_hbm.at[hidx_v], hrows_v, sems[p]),
            pltpu.async_copy(rel_hbm.at[ridx_v], wrows_v, sems[p]),
            pltpu.async_copy(z_hbm.at[tidx_v], trows_v, sems[p]),
        )

    for p in range(_NBUF):
        issue(p, p)

    for chunk in range(_NCHUNK):
        p = chunk % _NBUF
        for d in descs[p]:
            d.wait()
        hrows_v, wrows_v, trows_v = row_bufs[p]

        def group_body(g, carry):
            for i in range(_L):
                row = g * _L + i
                acc = jnp.zeros((_L,), jnp.float32)
                for s in range(DIM // _L):
                    hv = hrows_v[row, pl.ds(s * _L, _L)]
                    wv = wrows_v[row, pl.ds(s * _L, _L)]
                    tv = trows_v[row, pl.ds(s * _L, _L)]
                    acc = acc + hv * wv * tv
                stage_v[pl.ds(i * _L, _L)] = acc
            tot = jnp.zeros((_L,), jnp.float32)
            for j in range(_L):
                tot = tot + plsc.load_gather(stage_v, [lane16 + j])
            out_v[pl.ds(chunk * _C + g * _L, _L)] = tot
            return carry

        lax.fori_loop(0, _GROUPS, group_body, 0)

        nxt = chunk + _NBUF
        if nxt < _NCHUNK:
            issue(nxt, p)

    pltpu.sync_copy(out_v, out_hbm.at[pl.ds(base, _BPW)])


def _flat_body(z_hbm, hidx_hbm, ridx_hbm, tidx_hbm, rel_hbm, out_hbm,
               i0h, i0r, i0t, i1h, i1r, i1t,
               r0h, r0w, r0t, r1h, r1w, r1t,
               sem0, sem1, stage_v, out_v):
    idx_bufs = [(i0h, i0r, i0t), (i1h, i1r, i1t)]
    row_bufs = [(r0h, r0w, r0t), (r1h, r1w, r1t)]
    _sc_body(z_hbm, hidx_hbm, ridx_hbm, tidx_hbm, rel_hbm, out_hbm,
             idx_bufs, row_bufs, [sem0, sem1], stage_v, out_v)


@jax.jit
def _dist_mult_sc(z, h, r, t, rel_weight):
    mesh = plsc.VectorSubcoreMesh(core_axis_name="c", subcore_axis_name="s")
    f = functools.partial(
        pl.kernel,
        mesh=mesh,
        out_type=jax.ShapeDtypeStruct((BATCH,), jnp.float32),
        scratch_types=(
            [pltpu.VMEM((_C,), jnp.int32)] * 6
            + [pltpu.VMEM((_C, DIM), jnp.float32)] * 6
            + [pltpu.SemaphoreType.DMA] * 2
            + [pltpu.VMEM((_L * _L,), jnp.float32),
               pltpu.VMEM((_BPW,), jnp.float32)]
        ),
        compiler_params=pltpu.CompilerParams(needs_layout_passes=False),
    )(_flat_body)
    return f(z, h, r, t, rel_weight)


def kernel(z, triples, rel_weight):
    tri = triples.astype(jnp.int32)
    return _dist_mult_sc(z, tri[:, 0], tri[:, 1], tri[:, 2], rel_weight)
